# Initial kernel scaffold; baseline (speedup 1.0000x reference)
import jax
import jax.numpy as jnp
from jax.experimental import pallas as pl


def kernel(x, Wgate, Wg, Wu, Wd):
    raise NotImplementedError("write your pallas kernel here")


# speedup vs baseline: 1.3477x; 1.3477x over previous
import functools

import jax
import jax.numpy as jnp
from jax.experimental import pallas as pl
from jax.experimental.pallas import tpu as pltpu

TOPK = 2
BT = 256


def _gating_body(x_ref, wg_ref, logits_ref, topi_ref, topw_ref):
    lg = jnp.dot(x_ref[...], wg_ref[...], preferred_element_type=jnp.float32)
    logits_ref[...] = lg
    e = lg.shape[-1]
    col = jax.lax.broadcasted_iota(jnp.int32, lg.shape, 1)
    i1 = jnp.argmax(lg, axis=-1).astype(jnp.int32)
    m1 = jnp.max(lg, axis=-1)
    masked = jnp.where(col == i1[:, None], -jnp.inf, lg)
    i2 = jnp.argmax(masked, axis=-1).astype(jnp.int32)
    m2 = jnp.max(masked, axis=-1)
    a = jnp.exp(m2 - m1)
    w1 = 1.0 / (1.0 + a)
    topi_ref[...] = jnp.stack([i1, i2], axis=-1)
    topw_ref[...] = jnp.stack([w1, 1.0 - w1], axis=-1)


def _gating(x_flat, Wgate):
    t, h = x_flat.shape
    e = Wgate.shape[1]
    tg = 1024
    return pl.pallas_call(
        _gating_body,
        grid=(t // tg,),
        in_specs=[
            pl.BlockSpec((tg, h), lambda i: (i, 0)),
            pl.BlockSpec((h, e), lambda i: (0, 0)),
        ],
        out_specs=[
            pl.BlockSpec((tg, e), lambda i: (i, 0)),
            pl.BlockSpec((tg, TOPK), lambda i: (i, 0)),
            pl.BlockSpec((tg, TOPK), lambda i: (i, 0)),
        ],
        out_shape=[
            jax.ShapeDtypeStruct((t, e), jnp.float32),
            jax.ShapeDtypeStruct((t, TOPK), jnp.int32),
            jax.ShapeDtypeStruct((t, TOPK), jnp.float32),
        ],
    )(x_flat, Wgate)


def _ffn_body(be_ref, xs_ref, wrow_ref, wg_ref, wu_ref, wd_ref, ys_ref):
    del be_ref
    xb = xs_ref[...].astype(jnp.bfloat16)
    g = jnp.dot(xb, wg_ref[0], preferred_element_type=jnp.float32)
    u = jnp.dot(xb, wu_ref[0], preferred_element_type=jnp.float32)
    h1 = (g * jax.nn.sigmoid(g) * u).astype(jnp.bfloat16)
    o = jnp.dot(h1, wd_ref[0], preferred_element_type=jnp.float32)
    ys_ref[...] = o * wrow_ref[0, 0, :][:, None]


def _grouped_ffn(xs, wrow3d, Wg, Wu, Wd, block_expert):
    p, h = xs.shape
    _, _, f = Wg.shape
    nb = p // BT
    grid_spec = pltpu.PrefetchScalarGridSpec(
        num_scalar_prefetch=1,
        grid=(nb,),
        in_specs=[
            pl.BlockSpec((BT, h), lambda i, be: (i, 0)),
            pl.BlockSpec((1, 1, BT), lambda i, be: (i, 0, 0)),
            pl.BlockSpec((1, h, f), lambda i, be: (be[i], 0, 0)),
            pl.BlockSpec((1, h, f), lambda i, be: (be[i], 0, 0)),
            pl.BlockSpec((1, f, h), lambda i, be: (be[i], 0, 0)),
        ],
        out_specs=pl.BlockSpec((BT, h), lambda i, be: (i, 0)),
    )
    return pl.pallas_call(
        _ffn_body,
        grid_spec=grid_spec,
        out_shape=jax.ShapeDtypeStruct((p, h), jnp.float32),
    )(block_expert, xs, wrow3d, Wg, Wu, Wd)


@jax.jit
def kernel(x, Wgate, Wg, Wu, Wd):
    b, s, h = x.shape
    e = Wgate.shape[1]
    t = b * s
    a = t * TOPK
    p = a + e * BT
    nb = p // BT

    x_flat = x.reshape(t, h)
    logits, topi, topw = _gating(x_flat, Wgate)

    ef = topi.reshape(-1)
    oh = jax.nn.one_hot(ef, e, dtype=jnp.int32)
    cnt = oh.sum(axis=0)
    rank = jnp.take_along_axis(jnp.cumsum(oh, axis=0) - oh, ef[:, None], axis=1)[:, 0]
    cnt_pad = ((cnt + BT - 1) // BT) * BT
    ends = jnp.cumsum(cnt_pad)
    aligned_off = ends - cnt_pad
    slot = aligned_off[ef] + rank
    tok = jnp.arange(a, dtype=jnp.int32) // TOPK
    src = jnp.zeros((p,), jnp.int32).at[slot].set(tok)
    wrow = jnp.zeros((p,), jnp.float32).at[slot].set(topw.reshape(-1))
    pos = slot.reshape(t, TOPK)
    block_expert = jnp.minimum(
        jnp.searchsorted(ends, jnp.arange(nb, dtype=jnp.int32) * BT, side="right"),
        e - 1,
    ).astype(jnp.int32)

    xs = x_flat[src]

    ys = _grouped_ffn(
        xs,
        wrow.reshape(nb, 1, BT),
        Wg.astype(jnp.bfloat16),
        Wu.astype(jnp.bfloat16),
        Wd.astype(jnp.bfloat16),
        block_expert,
    )

    out = ys[pos[:, 0]] + ys[pos[:, 1]]

    return out.reshape(b, s, h), logits
